# Initial kernel scaffold; baseline (speedup 1.0000x reference)
import jax
import jax.numpy as jnp
from jax.experimental import pallas as pl


def kernel(x, edge_attr, edge_index, W_node, b_node, W_edge, b_edge, W1, b1, W2, b2, gamma, beta, Wl2, bl2, Wl3, bl3):
    raise NotImplementedError("write your pallas kernel here")


# speedup vs baseline: 3.2622x; 3.2622x over previous
import functools

import jax
import jax.numpy as jnp
from jax import lax
from jax.experimental import pallas as pl
from jax.experimental.pallas import tpu as pltpu
from jax.experimental.pallas import tpu_sc as plsc

N = 10000
E = 320000
DF = 128
DE = 16
H = 32
L = 3

NC = 2
NS = 16
NW = NC * NS
EPT = E // NW
CH = 80
NCH = EPT // CH
NP = 10240
RPT = NP // NS


def _edge_pass(h, ea, src, dst):
    mesh = plsc.VectorSubcoreMesh(core_axis_name="c", subcore_axis_name="s")

    @functools.partial(
        pl.kernel,
        out_type=jax.ShapeDtypeStruct((NC * NP, H), jnp.float32),
        mesh=mesh,
        scratch_types=[
            pltpu.VMEM((CH,), jnp.int32),
            pltpu.VMEM((CH,), jnp.int32),
            pltpu.VMEM((CH, H), jnp.float32),
            pltpu.VMEM((CH, H), jnp.float32),
            pltpu.VMEM((CH, H), jnp.float32),
            pltpu.VMEM((RPT, H), jnp.float32),
            pltpu.VMEM_SHARED((NP, H), jnp.float32),
            pltpu.SemaphoreType.DMA,
        ],
        compiler_params=pltpu.CompilerParams(use_tc_tiling_on_sc=False),
    )
    def edge_kernel(h_hbm, ea_hbm, src_hbm, dst_hbm, out_hbm,
                    src_v, dst_v, hsrc_v, ea_v, msg_v, buf_v, agg_sh, gsem):
        c = lax.axis_index("c")
        s = lax.axis_index("s")
        wid = c * NS + s
        base = wid * EPT

        zv = jnp.zeros((16,), jnp.float32)

        def zrow(r, carry):
            buf_v[r, pl.ds(0, 16)] = zv
            buf_v[r, pl.ds(16, 16)] = zv
            return carry

        lax.fori_loop(0, RPT, zrow, 0)
        pltpu.sync_copy(buf_v, agg_sh.at[pl.ds(s * RPT, RPT)])
        plsc.subcore_barrier()

        def chunk(j, carry):
            eoff = base + j * CH
            pltpu.sync_copy(src_hbm.at[pl.ds(eoff, CH)], src_v)
            pltpu.sync_copy(dst_hbm.at[pl.ds(eoff, CH)], dst_v)
            pltpu.sync_copy(ea_hbm.at[pl.ds(eoff, CH)], ea_v)
            pltpu.async_copy(h_hbm.at[src_v], hsrc_v, gsem).wait()

            def row(r, rc):
                msg_v[r, pl.ds(0, 16)] = jnp.maximum(
                    hsrc_v[r, pl.ds(0, 16)] + ea_v[r, pl.ds(0, 16)], 0.0)
                msg_v[r, pl.ds(16, 16)] = jnp.maximum(
                    hsrc_v[r, pl.ds(16, 16)] + ea_v[r, pl.ds(16, 16)], 0.0)
                return rc

            lax.fori_loop(0, CH, row, 0)
            pltpu.sync_copy(msg_v, agg_sh.at[dst_v], add=True)
            return carry

        lax.fori_loop(0, NCH, chunk, 0)
        plsc.subcore_barrier()

        pltpu.sync_copy(agg_sh.at[pl.ds(s * RPT, RPT)], buf_v)
        pltpu.sync_copy(buf_v, out_hbm.at[pl.ds(wid * RPT, RPT)])

    return edge_kernel(h, ea, src, dst)


def _proj_node(x, W, b):
    def body(x_ref, w_ref, b_ref, o_ref):
        o_ref[...] = jnp.dot(x_ref[...], w_ref[...],
                             preferred_element_type=jnp.float32) + b_ref[...]

    return pl.pallas_call(
        body, out_shape=jax.ShapeDtypeStruct((N, H), jnp.float32),
    )(x, W, b.reshape(1, H))


def _proj_edge(ea, W, b):
    RB = 8000
    G = E // RB

    def body(a_ref, w_ref, b_ref, o_ref):
        o_ref[...] = jnp.dot(a_ref[...], w_ref[...],
                             preferred_element_type=jnp.float32) + b_ref[...]

    return pl.pallas_call(
        body,
        grid=(G,),
        in_specs=[
            pl.BlockSpec((RB, DE), lambda i: (i, 0)),
            pl.BlockSpec((DE, H), lambda i: (0, 0)),
            pl.BlockSpec((1, H), lambda i: (0, 0)),
        ],
        out_specs=pl.BlockSpec((RB, H), lambda i: (i, 0)),
        out_shape=jax.ShapeDtypeStruct((E, H), jnp.float32),
    )(ea, W, b.reshape(1, H))


def _dense_layer(h, a0, a1, W1i, b1i, W2i, b2i, gi, bi):
    def body(h_ref, a0_ref, a1_ref, w1, b1, w2, b2, g, bt, o_ref):
        z = h_ref[...] + a0_ref[...] + a1_ref[...]
        t = jnp.maximum(
            jnp.dot(z, w1[...], preferred_element_type=jnp.float32) + b1[...], 0.0)
        z2 = jnp.dot(t, w2[...], preferred_element_type=jnp.float32) + b2[...]
        mu = jnp.mean(z2, axis=0, keepdims=True)
        var = jnp.mean((z2 - mu) ** 2, axis=0, keepdims=True)
        o_ref[...] = jnp.maximum(
            g[...] * (z2 - mu) * lax.rsqrt(var + 1e-5) + bt[...], 0.0)

    return pl.pallas_call(
        body, out_shape=jax.ShapeDtypeStruct((N, H), jnp.float32),
    )(h, a0, a1, W1i, b1i.reshape(1, 2 * H), W2i, b2i.reshape(1, H),
      gi.reshape(1, H), bi.reshape(1, H))


def _dense_final(h, a0, a1, W1i, b1i, W2i, b2i, gi, bi, Wl2, bl2, Wl3, bl3):
    def body(h_ref, a0_ref, a1_ref, w1, b1, w2, b2, g, bt,
             wl2, l2b, wl3, l3b, o_ref):
        z = h_ref[...] + a0_ref[...] + a1_ref[...]
        t = jnp.maximum(
            jnp.dot(z, w1[...], preferred_element_type=jnp.float32) + b1[...], 0.0)
        z2 = jnp.dot(t, w2[...], preferred_element_type=jnp.float32) + b2[...]
        mu = jnp.mean(z2, axis=0, keepdims=True)
        var = jnp.mean((z2 - mu) ** 2, axis=0, keepdims=True)
        hp = jnp.maximum(
            g[...] * (z2 - mu) * lax.rsqrt(var + 1e-5) + bt[...], 0.0)
        gx = jnp.sum(hp, axis=0, keepdims=True)
        p = jnp.maximum(
            jnp.dot(gx, wl2[...], preferred_element_type=jnp.float32) + l2b[...], 0.0)
        o_ref[...] = jnp.dot(p, wl3[...],
                             preferred_element_type=jnp.float32) + l3b[...]

    return pl.pallas_call(
        body, out_shape=jax.ShapeDtypeStruct((1, 1), jnp.float32),
    )(h, a0, a1, W1i, b1i.reshape(1, 2 * H), W2i, b2i.reshape(1, H),
      gi.reshape(1, H), bi.reshape(1, H),
      Wl2, bl2.reshape(1, H // 2), Wl3, bl3.reshape(1, 1))


def kernel(x, edge_attr, edge_index, W_node, b_node, W_edge, b_edge,
           W1, b1, W2, b2, gamma, beta, Wl2, bl2, Wl3, bl3):
    src = edge_index[0]
    dst = edge_index[1]
    h = _proj_node(x, W_node, b_node)
    ea = _proj_edge(edge_attr, W_edge, b_edge)
    for i in range(L):
        parts = _edge_pass(h, ea, src, dst)
        a0 = parts[:N]
        a1 = parts[NP:NP + N]
        if i < L - 1:
            h = _dense_layer(h, a0, a1, W1[i], b1[i], W2[i], b2[i],
                             gamma[i], beta[i])
        else:
            res = _dense_final(h, a0, a1, W1[i], b1[i], W2[i], b2[i],
                               gamma[i], beta[i], Wl2, bl2, Wl3, bl3)
    return res.reshape(1)


# speedup vs baseline: 5.8199x; 1.7841x over previous
import functools

import jax
import jax.numpy as jnp
from jax import lax
from jax.experimental import pallas as pl
from jax.experimental.pallas import tpu as pltpu
from jax.experimental.pallas import tpu_sc as plsc

N = 10000
E = 320000
DF = 128
DE = 16
H = 32
L = 3

NC = 2
NS = 16
NW = NC * NS
EPT = E // NW
CH = 128
NCH = EPT // CH
TAIL = EPT - NCH * CH
NP = 10240
RPT = NP // NS


def _edge_pass(h, ea, srcm, dstm, srct, dstt):
    mesh = plsc.VectorSubcoreMesh(core_axis_name="c", subcore_axis_name="s")

    @functools.partial(
        pl.kernel,
        out_type=jax.ShapeDtypeStruct((NC * NP, H), jnp.float32),
        mesh=mesh,
        scratch_types=[
            pltpu.VMEM((NCH, CH), jnp.int32),
            pltpu.VMEM((NCH, CH), jnp.int32),
            pltpu.VMEM((TAIL,), jnp.int32),
            pltpu.VMEM((TAIL,), jnp.int32),
            pltpu.VMEM((CH, H), jnp.float32),
            pltpu.VMEM((CH, H), jnp.float32),
            pltpu.VMEM((CH, H), jnp.float32),
            pltpu.VMEM((CH, H), jnp.float32),
            pltpu.VMEM((CH, H), jnp.float32),
            pltpu.VMEM((CH, H), jnp.float32),
            pltpu.VMEM((TAIL, H), jnp.float32),
            pltpu.VMEM((TAIL, H), jnp.float32),
            pltpu.VMEM((TAIL, H), jnp.float32),
            pltpu.VMEM((RPT, H), jnp.float32),
            pltpu.VMEM_SHARED((NP, H), jnp.float32),
            pltpu.SemaphoreType.DMA,
            pltpu.SemaphoreType.DMA,
            pltpu.SemaphoreType.DMA,
            pltpu.SemaphoreType.DMA,
            pltpu.SemaphoreType.DMA,
            pltpu.SemaphoreType.DMA,
            pltpu.SemaphoreType.DMA,
        ],
        compiler_params=pltpu.CompilerParams(use_tc_tiling_on_sc=False),
    )
    def edge_kernel(h_hbm, ea_hbm, srcm_hbm, dstm_hbm, srct_hbm, dstt_hbm,
                    out_hbm,
                    srcm_v, dstm_v, srct_v, dstt_v,
                    hs0, hs1, ea0, ea1, ms0, ms1,
                    hst, eat, mst, buf_v, agg_sh,
                    gsem0, gsem1, esem0, esem1, ssem0, ssem1, tsem):
        c = lax.axis_index("c")
        s = lax.axis_index("s")
        wid = c * NS + s
        base = wid * EPT

        zv = jnp.zeros((16,), jnp.float32)

        def zrow(r, carry):
            buf_v[r, pl.ds(0, 16)] = zv
            buf_v[r, pl.ds(16, 16)] = zv
            return carry

        lax.fori_loop(0, RPT, zrow, 0, unroll=8)
        pltpu.sync_copy(buf_v, agg_sh.at[pl.ds(s * RPT, RPT)])

        pltpu.sync_copy(srcm_hbm.at[wid], srcm_v)
        pltpu.sync_copy(dstm_hbm.at[wid], dstm_v)
        pltpu.sync_copy(srct_hbm.at[wid], srct_v)
        pltpu.sync_copy(dstt_hbm.at[wid], dstt_v)
        plsc.subcore_barrier()

        def issue(k, eab, hsb, esem, gsem):
            pltpu.async_copy(ea_hbm.at[pl.ds(base + k * CH, CH)], eab, esem)
            pltpu.async_copy(h_hbm.at[srcm_v.at[k]], hsb, gsem)

        def wait_loads(k, eab, hsb, esem, gsem):
            pltpu.make_async_copy(
                ea_hbm.at[pl.ds(base + k * CH, CH)], eab, esem).wait()
            pltpu.make_async_copy(h_hbm.at[srcm_v.at[k]], hsb, gsem).wait()

        def compute(eab, hsb, msb):
            def row(r, rc):
                msb[r, pl.ds(0, 16)] = jnp.maximum(
                    hsb[r, pl.ds(0, 16)] + eab[r, pl.ds(0, 16)], 0.0)
                msb[r, pl.ds(16, 16)] = jnp.maximum(
                    hsb[r, pl.ds(16, 16)] + eab[r, pl.ds(16, 16)], 0.0)
                return rc

            lax.fori_loop(0, CH, row, 0, unroll=8)

        issue(0, ea0, hs0, esem0, gsem0)
        issue(1, ea1, hs1, esem1, gsem1)

        def pair(i, carry):
            k0 = 2 * i
            k1 = 2 * i + 1

            wait_loads(k0, ea0, hs0, esem0, gsem0)

            @pl.when(i > 0)
            def _():
                pltpu.make_async_copy(
                    ms0, agg_sh.at[dstm_v.at[k0 - 2]], ssem0).wait()

            compute(ea0, hs0, ms0)

            @pl.when(k0 + 2 < NCH)
            def _():
                issue(k0 + 2, ea0, hs0, esem0, gsem0)

            pltpu.async_copy(ms0, agg_sh.at[dstm_v.at[k0]], ssem0, add=True)

            wait_loads(k1, ea1, hs1, esem1, gsem1)

            @pl.when(i > 0)
            def _():
                pltpu.make_async_copy(
                    ms1, agg_sh.at[dstm_v.at[k1 - 2]], ssem1).wait()

            compute(ea1, hs1, ms1)

            @pl.when(k1 + 2 < NCH)
            def _():
                issue(k1 + 2, ea1, hs1, esem1, gsem1)

            pltpu.async_copy(ms1, agg_sh.at[dstm_v.at[k1]], ssem1, add=True)
            return carry

        lax.fori_loop(0, NCH // 2, pair, 0)

        pltpu.make_async_copy(ms0, agg_sh.at[dstm_v.at[NCH - 2]], ssem0).wait()
        pltpu.make_async_copy(ms1, agg_sh.at[dstm_v.at[NCH - 1]], ssem1).wait()

        pltpu.sync_copy(ea_hbm.at[pl.ds(base + NCH * CH, TAIL)], eat)
        pltpu.async_copy(h_hbm.at[srct_v], hst, tsem).wait()

        def trow(r, rc):
            mst[r, pl.ds(0, 16)] = jnp.maximum(
                hst[r, pl.ds(0, 16)] + eat[r, pl.ds(0, 16)], 0.0)
            mst[r, pl.ds(16, 16)] = jnp.maximum(
                hst[r, pl.ds(16, 16)] + eat[r, pl.ds(16, 16)], 0.0)
            return rc

        lax.fori_loop(0, TAIL, trow, 0, unroll=8)
        pltpu.sync_copy(mst, agg_sh.at[dstt_v], add=True)

        plsc.subcore_barrier()
        pltpu.sync_copy(agg_sh.at[pl.ds(s * RPT, RPT)], buf_v)
        pltpu.sync_copy(buf_v, out_hbm.at[pl.ds(wid * RPT, RPT)])

    return edge_kernel(h, ea, srcm, dstm, srct, dstt)


def _proj_node(x, W, b):
    def body(x_ref, w_ref, b_ref, o_ref):
        o_ref[...] = jnp.dot(x_ref[...], w_ref[...],
                             preferred_element_type=jnp.float32) + b_ref[...]

    return pl.pallas_call(
        body, out_shape=jax.ShapeDtypeStruct((N, H), jnp.float32),
    )(x, W, b.reshape(1, H))


def _proj_edge(ea, W, b):
    RB = 8000
    G = E // RB

    def body(a_ref, w_ref, b_ref, o_ref):
        o_ref[...] = jnp.dot(a_ref[...], w_ref[...],
                             preferred_element_type=jnp.float32) + b_ref[...]

    return pl.pallas_call(
        body,
        grid=(G,),
        in_specs=[
            pl.BlockSpec((RB, DE), lambda i: (i, 0)),
            pl.BlockSpec((DE, H), lambda i: (0, 0)),
            pl.BlockSpec((1, H), lambda i: (0, 0)),
        ],
        out_specs=pl.BlockSpec((RB, H), lambda i: (i, 0)),
        out_shape=jax.ShapeDtypeStruct((E, H), jnp.float32),
    )(ea, W, b.reshape(1, H))


def _dense_layer(h, a0, a1, W1i, b1i, W2i, b2i, gi, bi):
    def body(h_ref, a0_ref, a1_ref, w1, b1, w2, b2, g, bt, o_ref):
        z = h_ref[...] + a0_ref[...] + a1_ref[...]
        t = jnp.maximum(
            jnp.dot(z, w1[...], preferred_element_type=jnp.float32) + b1[...], 0.0)
        z2 = jnp.dot(t, w2[...], preferred_element_type=jnp.float32) + b2[...]
        mu = jnp.mean(z2, axis=0, keepdims=True)
        var = jnp.mean((z2 - mu) ** 2, axis=0, keepdims=True)
        o_ref[...] = jnp.maximum(
            g[...] * (z2 - mu) * lax.rsqrt(var + 1e-5) + bt[...], 0.0)

    return pl.pallas_call(
        body, out_shape=jax.ShapeDtypeStruct((N, H), jnp.float32),
    )(h, a0, a1, W1i, b1i.reshape(1, 2 * H), W2i, b2i.reshape(1, H),
      gi.reshape(1, H), bi.reshape(1, H))


def _dense_final(h, a0, a1, W1i, b1i, W2i, b2i, gi, bi, Wl2, bl2, Wl3, bl3):
    def body(h_ref, a0_ref, a1_ref, w1, b1, w2, b2, g, bt,
             wl2, l2b, wl3, l3b, o_ref):
        z = h_ref[...] + a0_ref[...] + a1_ref[...]
        t = jnp.maximum(
            jnp.dot(z, w1[...], preferred_element_type=jnp.float32) + b1[...], 0.0)
        z2 = jnp.dot(t, w2[...], preferred_element_type=jnp.float32) + b2[...]
        mu = jnp.mean(z2, axis=0, keepdims=True)
        var = jnp.mean((z2 - mu) ** 2, axis=0, keepdims=True)
        hp = jnp.maximum(
            g[...] * (z2 - mu) * lax.rsqrt(var + 1e-5) + bt[...], 0.0)
        gx = jnp.sum(hp, axis=0, keepdims=True)
        p = jnp.maximum(
            jnp.dot(gx, wl2[...], preferred_element_type=jnp.float32) + l2b[...], 0.0)
        o_ref[...] = jnp.dot(p, wl3[...],
                             preferred_element_type=jnp.float32) + l3b[...]

    return pl.pallas_call(
        body, out_shape=jax.ShapeDtypeStruct((1, 1), jnp.float32),
    )(h, a0, a1, W1i, b1i.reshape(1, 2 * H), W2i, b2i.reshape(1, H),
      gi.reshape(1, H), bi.reshape(1, H),
      Wl2, bl2.reshape(1, H // 2), Wl3, bl3.reshape(1, 1))


def kernel(x, edge_attr, edge_index, W_node, b_node, W_edge, b_edge,
           W1, b1, W2, b2, gamma, beta, Wl2, bl2, Wl3, bl3):
    src2 = edge_index[0].reshape(NW, EPT)
    dst2 = edge_index[1].reshape(NW, EPT)
    srcm = src2[:, :NCH * CH].reshape(NW, NCH, CH)
    dstm = dst2[:, :NCH * CH].reshape(NW, NCH, CH)
    srct = src2[:, NCH * CH:]
    dstt = dst2[:, NCH * CH:]
    h = _proj_node(x, W_node, b_node)
    ea = _proj_edge(edge_attr, W_edge, b_edge)
    for i in range(L):
        parts = _edge_pass(h, ea, srcm, dstm, srct, dstt)
        a0 = parts[:N]
        a1 = parts[NP:NP + N]
        if i < L - 1:
            h = _dense_layer(h, a0, a1, W1[i], b1[i], W2[i], b2[i],
                             gamma[i], beta[i])
        else:
            res = _dense_final(h, a0, a1, W1[i], b1[i], W2[i], b2[i],
                               gamma[i], beta[i], Wl2, bl2, Wl3, bl3)
    return res.reshape(1)


# speedup vs baseline: 6.8962x; 1.1849x over previous
import functools

import jax
import jax.numpy as jnp
from jax import lax
from jax.experimental import pallas as pl
from jax.experimental.pallas import tpu as pltpu
from jax.experimental.pallas import tpu_sc as plsc

N = 10000
E = 320000
DF = 128
DE = 16
H = 32
L = 3

NC = 2
NS = 16
NW = NC * NS
EPT = E // NW
CH = 128
NCH = EPT // CH
TAIL = EPT - NCH * CH
NP = 10240
RPT = NP // NS


def _edge_pass(h, ea, srcm, dstm, srct, dstt):
    mesh = plsc.VectorSubcoreMesh(core_axis_name="c", subcore_axis_name="s")

    @functools.partial(
        pl.kernel,
        out_type=jax.ShapeDtypeStruct((NC * NP, H), jnp.float32),
        mesh=mesh,
        scratch_types=[
            pltpu.VMEM((NCH, CH), jnp.int32),
            pltpu.VMEM((NCH, CH), jnp.int32),
            pltpu.VMEM((TAIL,), jnp.int32),
            pltpu.VMEM((TAIL,), jnp.int32),
            pltpu.VMEM((CH, H), jnp.float32),
            pltpu.VMEM((CH, H), jnp.float32),
            pltpu.VMEM((CH // 4, 4 * H), jnp.float32),
            pltpu.VMEM((CH // 4, 4 * H), jnp.float32),
            pltpu.VMEM((CH, H), jnp.float32),
            pltpu.VMEM((CH, H), jnp.float32),
            pltpu.VMEM((TAIL, H), jnp.float32),
            pltpu.VMEM((TAIL // 4, 4 * H), jnp.float32),
            pltpu.VMEM((TAIL, H), jnp.float32),
            pltpu.VMEM((RPT, H), jnp.float32),
            pltpu.VMEM_SHARED((NP, H), jnp.float32),
            pltpu.SemaphoreType.DMA,
            pltpu.SemaphoreType.DMA,
            pltpu.SemaphoreType.DMA,
            pltpu.SemaphoreType.DMA,
            pltpu.SemaphoreType.DMA,
            pltpu.SemaphoreType.DMA,
            pltpu.SemaphoreType.DMA,
        ],
        compiler_params=pltpu.CompilerParams(use_tc_tiling_on_sc=False),
    )
    def edge_kernel(h_hbm, ea_hbm, srcm_hbm, dstm_hbm, srct_hbm, dstt_hbm,
                    out_hbm,
                    srcm_v, dstm_v, srct_v, dstt_v,
                    hs0, hs1, ea0, ea1, ms0, ms1,
                    hst, eat, mst, buf_v, agg_sh,
                    gsem0, gsem1, esem0, esem1, ssem0, ssem1, tsem):
        c = lax.axis_index("c")
        s = lax.axis_index("s")
        wid = c * NS + s
        base = wid * EPT

        zv = jnp.zeros((16,), jnp.float32)

        def zrow(r, carry):
            buf_v[r, pl.ds(0, 16)] = zv
            buf_v[r, pl.ds(16, 16)] = zv
            return carry

        lax.fori_loop(0, RPT, zrow, 0, unroll=8)
        pltpu.sync_copy(buf_v, agg_sh.at[pl.ds(s * RPT, RPT)])

        pltpu.sync_copy(srcm_hbm.at[wid], srcm_v)
        pltpu.sync_copy(dstm_hbm.at[wid], dstm_v)
        pltpu.sync_copy(srct_hbm.at[wid], srct_v)
        pltpu.sync_copy(dstt_hbm.at[wid], dstt_v)
        plsc.subcore_barrier()

        base4 = wid * (EPT // 4)

        def issue(k, eab, hsb, esem, gsem):
            pltpu.async_copy(
                ea_hbm.at[pl.ds(base4 + k * (CH // 4), CH // 4)], eab, esem)
            pltpu.async_copy(h_hbm.at[srcm_v.at[k]], hsb, gsem)

        def wait_loads(k, eab, hsb, esem, gsem):
            pltpu.make_async_copy(
                ea_hbm.at[pl.ds(base4 + k * (CH // 4), CH // 4)], eab,
                esem).wait()
            pltpu.make_async_copy(h_hbm.at[srcm_v.at[k]], hsb, gsem).wait()

        def compute(eab, hsb, msb):
            def row(q, rc):
                for cc in range(4):
                    r = 4 * q + cc
                    for hh in range(2):
                        msb[r, pl.ds(hh * 16, 16)] = jnp.maximum(
                            hsb[r, pl.ds(hh * 16, 16)]
                            + eab[q, pl.ds((cc * 2 + hh) * 16, 16)], 0.0)
                return rc

            lax.fori_loop(0, CH // 4, row, 0, unroll=2)

        issue(0, ea0, hs0, esem0, gsem0)
        issue(1, ea1, hs1, esem1, gsem1)

        def pair(i, carry):
            k0 = 2 * i
            k1 = 2 * i + 1

            wait_loads(k0, ea0, hs0, esem0, gsem0)

            @pl.when(i > 0)
            def _():
                pltpu.make_async_copy(
                    ms0, agg_sh.at[dstm_v.at[k0 - 2]], ssem0).wait()

            compute(ea0, hs0, ms0)

            @pl.when(k0 + 2 < NCH)
            def _():
                issue(k0 + 2, ea0, hs0, esem0, gsem0)

            pltpu.async_copy(ms0, agg_sh.at[dstm_v.at[k0]], ssem0, add=True)

            wait_loads(k1, ea1, hs1, esem1, gsem1)

            @pl.when(i > 0)
            def _():
                pltpu.make_async_copy(
                    ms1, agg_sh.at[dstm_v.at[k1 - 2]], ssem1).wait()

            compute(ea1, hs1, ms1)

            @pl.when(k1 + 2 < NCH)
            def _():
                issue(k1 + 2, ea1, hs1, esem1, gsem1)

            pltpu.async_copy(ms1, agg_sh.at[dstm_v.at[k1]], ssem1, add=True)
            return carry

        lax.fori_loop(0, NCH // 2, pair, 0)

        pltpu.make_async_copy(ms0, agg_sh.at[dstm_v.at[NCH - 2]], ssem0).wait()
        pltpu.make_async_copy(ms1, agg_sh.at[dstm_v.at[NCH - 1]], ssem1).wait()

        pltpu.sync_copy(
            ea_hbm.at[pl.ds(base4 + NCH * (CH // 4), TAIL // 4)], eat)
        pltpu.async_copy(h_hbm.at[srct_v], hst, tsem).wait()

        for q in range(TAIL // 4):
            for cc in range(4):
                r = 4 * q + cc
                for hh in range(2):
                    mst[r, pl.ds(hh * 16, 16)] = jnp.maximum(
                        hst[r, pl.ds(hh * 16, 16)]
                        + eat[q, pl.ds((cc * 2 + hh) * 16, 16)], 0.0)
        pltpu.sync_copy(mst, agg_sh.at[dstt_v], add=True)

        plsc.subcore_barrier()
        pltpu.sync_copy(agg_sh.at[pl.ds(s * RPT, RPT)], buf_v)
        pltpu.sync_copy(buf_v, out_hbm.at[pl.ds(wid * RPT, RPT)])

    return edge_kernel(h, ea, srcm, dstm, srct, dstt)


def _proj_node(x, W, b):
    def body(x_ref, w_ref, b_ref, o_ref):
        o_ref[...] = jnp.dot(x_ref[...], w_ref[...],
                             preferred_element_type=jnp.float32) + b_ref[...]

    return pl.pallas_call(
        body, out_shape=jax.ShapeDtypeStruct((N, H), jnp.float32),
    )(x, W, b.reshape(1, H))


def _proj_edge(eaT, W, b):
    RB = 12800
    G = E // RB

    RB4 = RB // 4

    def body(a_ref, w_ref, b_ref, o_ref):
        y = lax.dot_general(a_ref[...], w_ref[...],
                            (((0,), (0,)), ((), ())),
                            preferred_element_type=jnp.float32) + b_ref[...]
        o_ref[...] = jnp.concatenate(
            [y[0:RB4], y[RB4:2 * RB4], y[2 * RB4:3 * RB4], y[3 * RB4:]],
            axis=1)

    return pl.pallas_call(
        body,
        grid=(G,),
        in_specs=[
            pl.BlockSpec((DE, RB), lambda i: (0, i)),
            pl.BlockSpec((DE, H), lambda i: (0, 0)),
            pl.BlockSpec((1, H), lambda i: (0, 0)),
        ],
        out_specs=pl.BlockSpec((RB // 4, 4 * H), lambda i: (i, 0)),
        out_shape=jax.ShapeDtypeStruct((E // 4, 4 * H), jnp.float32),
    )(eaT, W, b.reshape(1, H))


def _dense_layer(h, a0, a1, W1i, b1i, W2i, b2i, gi, bi):
    def body(h_ref, a0_ref, a1_ref, w1, b1, w2, b2, g, bt, o_ref):
        z = h_ref[...] + a0_ref[...] + a1_ref[...]
        t = jnp.maximum(
            jnp.dot(z, w1[...], preferred_element_type=jnp.float32) + b1[...], 0.0)
        z2 = jnp.dot(t, w2[...], preferred_element_type=jnp.float32) + b2[...]
        mu = jnp.mean(z2, axis=0, keepdims=True)
        var = jnp.mean((z2 - mu) ** 2, axis=0, keepdims=True)
        o_ref[...] = jnp.maximum(
            g[...] * (z2 - mu) * lax.rsqrt(var + 1e-5) + bt[...], 0.0)

    return pl.pallas_call(
        body, out_shape=jax.ShapeDtypeStruct((N, H), jnp.float32),
    )(h, a0, a1, W1i, b1i.reshape(1, 2 * H), W2i, b2i.reshape(1, H),
      gi.reshape(1, H), bi.reshape(1, H))


def _dense_final(h, a0, a1, W1i, b1i, W2i, b2i, gi, bi, Wl2, bl2, Wl3, bl3):
    def body(h_ref, a0_ref, a1_ref, w1, b1, w2, b2, g, bt,
             wl2, l2b, wl3, l3b, o_ref):
        z = h_ref[...] + a0_ref[...] + a1_ref[...]
        t = jnp.maximum(
            jnp.dot(z, w1[...], preferred_element_type=jnp.float32) + b1[...], 0.0)
        z2 = jnp.dot(t, w2[...], preferred_element_type=jnp.float32) + b2[...]
        mu = jnp.mean(z2, axis=0, keepdims=True)
        var = jnp.mean((z2 - mu) ** 2, axis=0, keepdims=True)
        hp = jnp.maximum(
            g[...] * (z2 - mu) * lax.rsqrt(var + 1e-5) + bt[...], 0.0)
        gx = jnp.sum(hp, axis=0, keepdims=True)
        p = jnp.maximum(
            jnp.dot(gx, wl2[...], preferred_element_type=jnp.float32) + l2b[...], 0.0)
        o_ref[...] = jnp.dot(p, wl3[...],
                             preferred_element_type=jnp.float32) + l3b[...]

    return pl.pallas_call(
        body, out_shape=jax.ShapeDtypeStruct((1, 1), jnp.float32),
    )(h, a0, a1, W1i, b1i.reshape(1, 2 * H), W2i, b2i.reshape(1, H),
      gi.reshape(1, H), bi.reshape(1, H),
      Wl2, bl2.reshape(1, H // 2), Wl3, bl3.reshape(1, 1))


def kernel(x, edge_attr, edge_index, W_node, b_node, W_edge, b_edge,
           W1, b1, W2, b2, gamma, beta, Wl2, bl2, Wl3, bl3):
    PRB, PNB = 12800, E // 12800
    src2 = edge_index[0].reshape(PNB, 4, PRB // 4).transpose(0, 2, 1).reshape(NW, EPT)
    dst2 = edge_index[1].reshape(PNB, 4, PRB // 4).transpose(0, 2, 1).reshape(NW, EPT)
    srcm = src2[:, :NCH * CH].reshape(NW, NCH, CH)
    dstm = dst2[:, :NCH * CH].reshape(NW, NCH, CH)
    srct = src2[:, NCH * CH:]
    dstt = dst2[:, NCH * CH:]
    h = _proj_node(x, W_node, b_node)
    ea = _proj_edge(edge_attr.T, W_edge, b_edge)
    for i in range(L):
        parts = _edge_pass(h, ea, srcm, dstm, srct, dstt)
        a0 = parts[:N]
        a1 = parts[NP:NP + N]
        if i < L - 1:
            h = _dense_layer(h, a0, a1, W1[i], b1[i], W2[i], b2[i],
                             gamma[i], beta[i])
        else:
            res = _dense_final(h, a0, a1, W1[i], b1[i], W2[i], b2[i],
                               gamma[i], beta[i], Wl2, bl2, Wl3, bl3)
    return res.reshape(1)


# speedup vs baseline: 7.9083x; 1.1468x over previous
import functools

import jax
import jax.numpy as jnp
from jax import lax
from jax.experimental import pallas as pl
from jax.experimental.pallas import tpu as pltpu
from jax.experimental.pallas import tpu_sc as plsc

N = 10000
E = 320000
DF = 128
DE = 16
H = 32
L = 3

NC = 2
NS = 16
NW = NC * NS
EPT = E // NW
CH = 128
NCH = EPT // CH
TAIL = EPT - NCH * CH
NP = 10240
RPT = NP // NS
PRB = 12800
PRB4 = PRB // 4


def _edge_pass(h, ea, srcm, dstm, srct, dstt):
    mesh = plsc.VectorSubcoreMesh(core_axis_name="c", subcore_axis_name="s")

    @functools.partial(
        pl.kernel,
        out_type=jax.ShapeDtypeStruct((NC * NP, H), jnp.float32),
        mesh=mesh,
        scratch_types=[
            pltpu.VMEM((NCH, CH), jnp.int32),
            pltpu.VMEM((NCH, CH), jnp.int32),
            pltpu.VMEM((TAIL,), jnp.int32),
            pltpu.VMEM((TAIL,), jnp.int32),
            pltpu.VMEM((CH, H), jnp.float32),
            pltpu.VMEM((CH, H), jnp.float32),
            pltpu.VMEM((CH, H), jnp.float32),
            pltpu.VMEM((CH, H), jnp.float32),
            pltpu.VMEM((CH, H), jnp.float32),
            pltpu.VMEM((CH, H), jnp.float32),
            pltpu.VMEM((TAIL, H), jnp.float32),
            pltpu.VMEM((TAIL, H), jnp.float32),
            pltpu.VMEM((TAIL, H), jnp.float32),
            pltpu.VMEM((RPT, H), jnp.float32),
            pltpu.VMEM_SHARED((NP, H), jnp.float32),
            pltpu.SemaphoreType.DMA,
            pltpu.SemaphoreType.DMA,
            pltpu.SemaphoreType.DMA,
            pltpu.SemaphoreType.DMA,
            pltpu.SemaphoreType.DMA,
            pltpu.SemaphoreType.DMA,
            pltpu.SemaphoreType.DMA,
        ],
        compiler_params=pltpu.CompilerParams(use_tc_tiling_on_sc=False),
    )
    def edge_kernel(h_hbm, ea_hbm, srcm_hbm, dstm_hbm, srct_hbm, dstt_hbm,
                    out_hbm,
                    srcm_v, dstm_v, srct_v, dstt_v,
                    hs0, hs1, ea0, ea1, ms0, ms1,
                    hst, eat, mst, buf_v, agg_sh,
                    gsem0, gsem1, esem0, esem1, ssem0, ssem1, tsem):
        c = lax.axis_index("c")
        s = lax.axis_index("s")
        wid = c * NS + s
        base = wid * EPT

        zv = jnp.zeros((16,), jnp.float32)

        def zrow(r, carry):
            buf_v[r, pl.ds(0, 16)] = zv
            buf_v[r, pl.ds(16, 16)] = zv
            return carry

        lax.fori_loop(0, RPT, zrow, 0, unroll=8)
        pltpu.sync_copy(buf_v, agg_sh.at[pl.ds(s * RPT, RPT)])

        pltpu.sync_copy(srcm_hbm.at[wid], srcm_v)
        pltpu.sync_copy(dstm_hbm.at[wid], dstm_v)
        pltpu.sync_copy(srct_hbm.at[wid], srct_v)
        pltpu.sync_copy(dstt_hbm.at[wid], dstt_v)
        plsc.subcore_barrier()

        def ea_slice(e0):
            i = e0 // PRB
            rem = e0 - i * PRB
            cb = rem // PRB4
            row0 = i * PRB4 + (rem - cb * PRB4)
            return row0, cb * H

        def issue(k, eab, hsb, esem, gsem):
            row0, col0 = ea_slice(base + k * CH)
            pltpu.async_copy(
                ea_hbm.at[pl.ds(row0, CH), pl.ds(col0, H)], eab, esem)
            pltpu.async_copy(h_hbm.at[srcm_v.at[k]], hsb, gsem)

        def wait_loads(k, eab, hsb, esem, gsem):
            row0, col0 = ea_slice(base + k * CH)
            pltpu.make_async_copy(
                ea_hbm.at[pl.ds(row0, CH), pl.ds(col0, H)], eab, esem).wait()
            pltpu.make_async_copy(h_hbm.at[srcm_v.at[k]], hsb, gsem).wait()

        def compute(eab, hsb, msb):
            def row(r, rc):
                msb[r, pl.ds(0, 16)] = jnp.maximum(
                    hsb[r, pl.ds(0, 16)] + eab[r, pl.ds(0, 16)], 0.0)
                msb[r, pl.ds(16, 16)] = jnp.maximum(
                    hsb[r, pl.ds(16, 16)] + eab[r, pl.ds(16, 16)], 0.0)
                return rc

            lax.fori_loop(0, CH, row, 0, unroll=8)

        issue(0, ea0, hs0, esem0, gsem0)
        issue(1, ea1, hs1, esem1, gsem1)

        def pair(i, carry):
            k0 = 2 * i
            k1 = 2 * i + 1

            wait_loads(k0, ea0, hs0, esem0, gsem0)

            @pl.when(i > 0)
            def _():
                pltpu.make_async_copy(
                    ms0, agg_sh.at[dstm_v.at[k0 - 2]], ssem0).wait()

            compute(ea0, hs0, ms0)

            @pl.when(k0 + 2 < NCH)
            def _():
                issue(k0 + 2, ea0, hs0, esem0, gsem0)

            pltpu.async_copy(ms0, agg_sh.at[dstm_v.at[k0]], ssem0, add=True)

            wait_loads(k1, ea1, hs1, esem1, gsem1)

            @pl.when(i > 0)
            def _():
                pltpu.make_async_copy(
                    ms1, agg_sh.at[dstm_v.at[k1 - 2]], ssem1).wait()

            compute(ea1, hs1, ms1)

            @pl.when(k1 + 2 < NCH)
            def _():
                issue(k1 + 2, ea1, hs1, esem1, gsem1)

            pltpu.async_copy(ms1, agg_sh.at[dstm_v.at[k1]], ssem1, add=True)
            return carry

        lax.fori_loop(0, NCH // 2, pair, 0)

        pltpu.make_async_copy(ms0, agg_sh.at[dstm_v.at[NCH - 2]], ssem0).wait()
        pltpu.make_async_copy(ms1, agg_sh.at[dstm_v.at[NCH - 1]], ssem1).wait()

        trow0, tcol0 = ea_slice(base + NCH * CH)
        pltpu.sync_copy(
            ea_hbm.at[pl.ds(trow0, TAIL), pl.ds(tcol0, H)], eat)
        pltpu.async_copy(h_hbm.at[srct_v], hst, tsem).wait()

        for r in range(TAIL):
            for hh in range(2):
                mst[r, pl.ds(hh * 16, 16)] = jnp.maximum(
                    hst[r, pl.ds(hh * 16, 16)]
                    + eat[r, pl.ds(hh * 16, 16)], 0.0)
        pltpu.sync_copy(mst, agg_sh.at[dstt_v], add=True)

        plsc.subcore_barrier()
        pltpu.sync_copy(agg_sh.at[pl.ds(s * RPT, RPT)], buf_v)
        pltpu.sync_copy(buf_v, out_hbm.at[pl.ds(wid * RPT, RPT)])

    return edge_kernel(h, ea, srcm, dstm, srct, dstt)


def _proj_node(x, W, b):
    def body(x_ref, w_ref, b_ref, o_ref):
        o_ref[...] = jnp.dot(x_ref[...], w_ref[...],
                             preferred_element_type=jnp.float32) + b_ref[...]

    return pl.pallas_call(
        body, out_shape=jax.ShapeDtypeStruct((N, H), jnp.float32),
    )(x, W, b.reshape(1, H))


def _proj_edge(eaT, W, b):
    RB = PRB
    G = E // RB

    RB4 = RB // 4

    def body(a_ref, w_ref, b_ref, o_ref):
        y = lax.dot_general(a_ref[...], w_ref[...],
                            (((0,), (0,)), ((), ())),
                            preferred_element_type=jnp.float32) + b_ref[...]
        o_ref[...] = jnp.concatenate(
            [y[0:RB4], y[RB4:2 * RB4], y[2 * RB4:3 * RB4], y[3 * RB4:]],
            axis=1)

    return pl.pallas_call(
        body,
        grid=(G,),
        in_specs=[
            pl.BlockSpec((DE, RB), lambda i: (0, i)),
            pl.BlockSpec((DE, H), lambda i: (0, 0)),
            pl.BlockSpec((1, H), lambda i: (0, 0)),
        ],
        out_specs=pl.BlockSpec((RB // 4, 4 * H), lambda i: (i, 0)),
        out_shape=jax.ShapeDtypeStruct((E // 4, 4 * H), jnp.float32),
    )(eaT, W, b.reshape(1, H))


def _dense_layer(h, a0, a1, W1i, b1i, W2i, b2i, gi, bi):
    def body(h_ref, a0_ref, a1_ref, w1, b1, w2, b2, g, bt, o_ref):
        z = h_ref[...] + a0_ref[...] + a1_ref[...]
        t = jnp.maximum(
            jnp.dot(z, w1[...], preferred_element_type=jnp.float32) + b1[...], 0.0)
        z2 = jnp.dot(t, w2[...], preferred_element_type=jnp.float32) + b2[...]
        mu = jnp.mean(z2, axis=0, keepdims=True)
        var = jnp.mean((z2 - mu) ** 2, axis=0, keepdims=True)
        o_ref[...] = jnp.maximum(
            g[...] * (z2 - mu) * lax.rsqrt(var + 1e-5) + bt[...], 0.0)

    return pl.pallas_call(
        body, out_shape=jax.ShapeDtypeStruct((N, H), jnp.float32),
    )(h, a0, a1, W1i, b1i.reshape(1, 2 * H), W2i, b2i.reshape(1, H),
      gi.reshape(1, H), bi.reshape(1, H))


def _dense_final(h, a0, a1, W1i, b1i, W2i, b2i, gi, bi, Wl2, bl2, Wl3, bl3):
    def body(h_ref, a0_ref, a1_ref, w1, b1, w2, b2, g, bt,
             wl2, l2b, wl3, l3b, o_ref):
        z = h_ref[...] + a0_ref[...] + a1_ref[...]
        t = jnp.maximum(
            jnp.dot(z, w1[...], preferred_element_type=jnp.float32) + b1[...], 0.0)
        z2 = jnp.dot(t, w2[...], preferred_element_type=jnp.float32) + b2[...]
        mu = jnp.mean(z2, axis=0, keepdims=True)
        var = jnp.mean((z2 - mu) ** 2, axis=0, keepdims=True)
        hp = jnp.maximum(
            g[...] * (z2 - mu) * lax.rsqrt(var + 1e-5) + bt[...], 0.0)
        gx = jnp.sum(hp, axis=0, keepdims=True)
        p = jnp.maximum(
            jnp.dot(gx, wl2[...], preferred_element_type=jnp.float32) + l2b[...], 0.0)
        o_ref[...] = jnp.dot(p, wl3[...],
                             preferred_element_type=jnp.float32) + l3b[...]

    return pl.pallas_call(
        body, out_shape=jax.ShapeDtypeStruct((1, 1), jnp.float32),
    )(h, a0, a1, W1i, b1i.reshape(1, 2 * H), W2i, b2i.reshape(1, H),
      gi.reshape(1, H), bi.reshape(1, H),
      Wl2, bl2.reshape(1, H // 2), Wl3, bl3.reshape(1, 1))


def kernel(x, edge_attr, edge_index, W_node, b_node, W_edge, b_edge,
           W1, b1, W2, b2, gamma, beta, Wl2, bl2, Wl3, bl3):
    src2 = edge_index[0].reshape(NW, EPT)
    dst2 = edge_index[1].reshape(NW, EPT)
    srcm = src2[:, :NCH * CH].reshape(NW, NCH, CH)
    dstm = dst2[:, :NCH * CH].reshape(NW, NCH, CH)
    srct = src2[:, NCH * CH:]
    dstt = dst2[:, NCH * CH:]
    h = _proj_node(x, W_node, b_node)
    ea = _proj_edge(edge_attr.T, W_edge, b_edge)
    for i in range(L):
        parts = _edge_pass(h, ea, srcm, dstm, srct, dstt)
        a0 = parts[:N]
        a1 = parts[NP:NP + N]
        if i < L - 1:
            h = _dense_layer(h, a0, a1, W1[i], b1[i], W2[i], b2[i],
                             gamma[i], beta[i])
        else:
            res = _dense_final(h, a0, a1, W1[i], b1[i], W2[i], b2[i],
                               gamma[i], beta[i], Wl2, bl2, Wl3, bl3)
    return res.reshape(1)
